# Initial kernel scaffold; baseline (speedup 1.0000x reference)
import jax
import jax.numpy as jnp
from jax.experimental import pallas as pl


def kernel(x, edge_index, batch, target_encoding, W1, att_src1, att_dst1, b1, W2, att_src2, att_dst2, b2, fc_g1_w, fc_g1_b, emb_xt, conv1_w, conv1_b, bn1_g, bn1_b, conv2_w, conv2_b, bn2_g, bn2_b, conv3_w, conv3_b, bn3_g, bn3_b, fc_xt_w, fc_xt_b, bnf_g, bnf_b, fc1_w, fc1_b, fc2_w, fc2_b, out_w, out_b):
    raise NotImplementedError("write your pallas kernel here")


# speedup vs baseline: 4.9850x; 4.9850x over previous
import functools
import jax
import jax.numpy as jnp
from jax import lax
from jax.experimental import pallas as pl
from jax.experimental.pallas import tpu as pltpu
from jax.experimental.pallas import tpu_sc as plsc

_N = 10000
_NP = 10240
_E = 160000
_B = 128
_L = 1000
_F = 78
_FP = 80
_H = 10
_OD = 128
_ODP = 144
_ER = _E + _N
_EPAD = 172032
_CH = 128
_NT = 16
_RPT = _NP // 16

_f32 = jnp.float32
_i32 = jnp.int32
_PREC = lax.Precision.DEFAULT
_PREC_HI = lax.Precision.HIGHEST


def _lane_bcast(v, k):
    return lax.gather(
        v, jnp.full((16, 1), k, _i32),
        lax.GatherDimensionNumbers(offset_dims=(), collapsed_slice_dims=(0,),
                                   start_index_map=(0,)),
        (1,), mode=lax.GatherScatterMode.PROMISE_IN_BOUNDS)


def _tca_body(x_ref, w_ref, as_ref, ad_ref, h_out, aso, ado):
    xb = x_ref[...]
    h = jnp.dot(xb, w_ref[0], preferred_element_type=_f32, precision=_PREC)
    lane = lax.broadcasted_iota(_i32, h.shape, 1)
    h = jnp.where(lane == _F, 1.0, h)
    h_out[0] = h
    aso[0, 0] = jnp.sum(h * as_ref[0], axis=1)
    ado[0, 0] = jnp.sum(h * ad_ref[0], axis=1)


def _tca(x, w1h, asw, adw):
    bn = 1024
    return pl.pallas_call(
        _tca_body,
        grid=(_H, _NP // bn),
        in_specs=[
            pl.BlockSpec((bn, _F), lambda h, n: (n, 0)),
            pl.BlockSpec((1, _F, _FP), lambda h, n: (h, 0, 0)),
            pl.BlockSpec((1, 1, _FP), lambda h, n: (h, 0, 0)),
            pl.BlockSpec((1, 1, _FP), lambda h, n: (h, 0, 0)),
        ],
        out_specs=[
            pl.BlockSpec((1, bn, _FP), lambda h, n: (h, n, 0)),
            pl.BlockSpec((1, 1, bn), lambda h, n: (h, 0, n)),
            pl.BlockSpec((1, 1, bn), lambda h, n: (h, 0, n)),
        ],
        out_shape=[
            jax.ShapeDtypeStruct((_H, _NP, _FP), _f32),
            jax.ShapeDtypeStruct((_H, 1, _NP), _f32),
            jax.ShapeDtypeStruct((_H, 1, _NP), _f32),
        ],
    )(x, w1h, asw, adw)


def _sc1_body(h_hbm, as_hbm, ad_hbm, src_hbm, dst_hbm, zero_hbm, out_hbm,
              acc, idx_s, idx_d, idx_g, rows, asb, adb):
    c = lax.axis_index("c")
    s = lax.axis_index("s")
    nchunk = _EPAD // _NT // _CH

    def head_body(hi, _):
        head = hi * 2 + c
        pltpu.sync_copy(zero_hbm, acc.at[pl.ds(s * _RPT, _RPT)])
        plsc.subcore_barrier()

        def chunk_body(ci, _):
            off = s * (_EPAD // _NT) + ci * _CH
            pltpu.sync_copy(src_hbm.at[pl.ds(off, _CH)], idx_s)
            pltpu.sync_copy(dst_hbm.at[pl.ds(off, _CH)], idx_d)
            hoff = head * _NP
            for j in range(_CH // 16):
                sl = pl.ds(j * 16, 16)
                idx_g[sl] = idx_s[sl] + hoff
            pltpu.sync_copy(h_hbm.at[idx_g], rows)
            pltpu.sync_copy(as_hbm.at[idx_g], asb)
            for j in range(_CH // 16):
                sl = pl.ds(j * 16, 16)
                idx_g[sl] = idx_d[sl] + hoff
            pltpu.sync_copy(ad_hbm.at[idx_g], adb)
            for g in range(_CH // 16):
                sl = pl.ds(g * 16, 16)
                e = asb[sl] + adb[sl]
                e = jnp.maximum(e, 0.2 * e)
                w = jnp.exp(e)
                ei = off + g * 16 + lax.iota(_i32, 16)
                wg = jnp.where(ei < _ER, w, 0.0)
                for k in range(16):
                    wv = _lane_bcast(wg, k)
                    kk = g * 16 + k
                    for q in range(_FP // 16):
                        sl2 = pl.ds(q * 16, 16)
                        rows[kk, sl2] = rows[kk, sl2] * wv
            pltpu.sync_copy(rows, acc.at[idx_d], add=True)
            return _

        lax.fori_loop(0, nchunk, chunk_body, None)
        plsc.subcore_barrier()
        pltpu.sync_copy(acc.at[pl.ds(s * _RPT, _RPT)],
                        out_hbm.at[pl.ds(s * _RPT, _RPT), head])
        plsc.subcore_barrier()
        return _

    lax.fori_loop(0, _H // 2, head_body, None)


def _sc1(h_flat, as_flat, ad_flat, srcp, dstp, zeros1):
    mesh = plsc.VectorSubcoreMesh(core_axis_name="c", subcore_axis_name="s")
    return pl.kernel(
        _sc1_body,
        out_type=jax.ShapeDtypeStruct((_NP, _H, _FP), _f32),
        mesh=mesh,
        compiler_params=pltpu.CompilerParams(use_tc_tiling_on_sc=False),
        scratch_types=[
            pltpu.VMEM_SHARED((_NP, _FP), _f32),
            pltpu.VMEM((_CH,), _i32),
            pltpu.VMEM((_CH,), _i32),
            pltpu.VMEM((_CH,), _i32),
            pltpu.VMEM((_CH, _FP), _f32),
            pltpu.VMEM((_CH,), _f32),
            pltpu.VMEM((_CH,), _f32),
        ],
    )(h_flat, as_flat, ad_flat, srcp, dstp, zeros1)


def _tcb_body(v_ref, w2_ref, b1_ref, a2s_ref, a2d_ref, h2o, aso, ado):
    bn = v_ref.shape[0]
    v = v_ref[...]
    den = v[:, :, _F:_F + 1] + 1e-16
    v = (v / den).reshape(bn, _H * _FP)
    v = v + b1_ref[0]
    x2 = jnp.where(v > 0, v, jnp.exp(jnp.minimum(v, 0.0)) - 1.0)
    h2 = jnp.dot(x2, w2_ref[...], preferred_element_type=_f32, precision=_PREC)
    ones = jnp.ones((bn, 1), _f32)
    zer = jnp.zeros((bn, _ODP - _OD - 1), _f32)
    h2o[...] = jnp.concatenate([h2, ones, zer], axis=1)
    aso[0] = jnp.sum(h2 * a2s_ref[...], axis=1)
    ado[0] = jnp.sum(h2 * a2d_ref[...], axis=1)


def _tcb(out1, w2p, b1p, a2s, a2d):
    bn = 1024
    return pl.pallas_call(
        _tcb_body,
        grid=(_NP // bn,),
        in_specs=[
            pl.BlockSpec((bn, _H, _FP), lambda n: (n, 0, 0)),
            pl.BlockSpec((_H * _FP, _OD), lambda n: (0, 0)),
            pl.BlockSpec((1, _H * _FP), lambda n: (0, 0)),
            pl.BlockSpec((1, _OD), lambda n: (0, 0)),
            pl.BlockSpec((1, _OD), lambda n: (0, 0)),
        ],
        out_specs=[
            pl.BlockSpec((bn, _ODP), lambda n: (n, 0)),
            pl.BlockSpec((1, bn), lambda n: (0, n)),
            pl.BlockSpec((1, bn), lambda n: (0, n)),
        ],
        out_shape=[
            jax.ShapeDtypeStruct((_NP, _ODP), _f32),
            jax.ShapeDtypeStruct((1, _NP), _f32),
            jax.ShapeDtypeStruct((1, _NP), _f32),
        ],
    )(out1, w2p, b1p, a2s, a2d)


def _sc2_body(h_hbm, as_hbm, ad_hbm, src_hbm, dst_hbm, zero_hbm, out_hbm,
              acc, idx_s, idx_d, rows, asb, adb):
    c = lax.axis_index("c")
    s = lax.axis_index("s")
    per_tile = _EPAD // 32
    nchunk = per_tile // _CH

    pltpu.sync_copy(zero_hbm, acc.at[pl.ds(s * _RPT, _RPT)])
    plsc.subcore_barrier()

    def chunk_body(ci, _):
        off = (c * _NT + s) * per_tile + ci * _CH
        pltpu.sync_copy(src_hbm.at[pl.ds(off, _CH)], idx_s)
        pltpu.sync_copy(dst_hbm.at[pl.ds(off, _CH)], idx_d)
        pltpu.sync_copy(h_hbm.at[idx_s], rows)
        pltpu.sync_copy(as_hbm.at[idx_s], asb)
        pltpu.sync_copy(ad_hbm.at[idx_d], adb)
        for g in range(_CH // 16):
            sl = pl.ds(g * 16, 16)
            e = asb[sl] + adb[sl]
            e = jnp.maximum(e, 0.2 * e)
            w = jnp.exp(e)
            ei = off + g * 16 + lax.iota(_i32, 16)
            wg = jnp.where(ei < _ER, w, 0.0)
            for k in range(16):
                wv = _lane_bcast(wg, k)
                kk = g * 16 + k
                for q in range(_ODP // 16):
                    sl2 = pl.ds(q * 16, 16)
                    rows[kk, sl2] = rows[kk, sl2] * wv
        pltpu.sync_copy(rows, acc.at[idx_d], add=True)
        return _

    lax.fori_loop(0, nchunk, chunk_body, None)
    plsc.subcore_barrier()
    pltpu.sync_copy(acc.at[pl.ds(s * _RPT, _RPT)],
                    out_hbm.at[c, pl.ds(s * _RPT, _RPT)])


def _sc2(h2aug, as2, ad2, srcp, dstp, zeros2):
    mesh = plsc.VectorSubcoreMesh(core_axis_name="c", subcore_axis_name="s")
    return pl.kernel(
        _sc2_body,
        out_type=jax.ShapeDtypeStruct((2, _NP, _ODP), _f32),
        mesh=mesh,
        compiler_params=pltpu.CompilerParams(use_tc_tiling_on_sc=False),
        scratch_types=[
            pltpu.VMEM_SHARED((_NP, _ODP), _f32),
            pltpu.VMEM((_CH,), _i32),
            pltpu.VMEM((_CH,), _i32),
            pltpu.VMEM((_CH, _ODP), _f32),
            pltpu.VMEM((_CH,), _f32),
            pltpu.VMEM((_CH,), _f32),
        ],
    )(h2aug, as2, ad2, srcp, dstp, zeros2)


def _tcc_body(p_ref, bid_ref, b2_ref, hg_ref):
    i = pl.program_id(0)
    bn = p_ref.shape[1]
    p = p_ref[...]
    num = p[0, :, :_OD] + p[1, :, :_OD]
    den = p[0, :, _OD:_OD + 1] + p[1, :, _OD:_OD + 1] + 1e-16
    h2 = jnp.maximum(num / den + b2_ref[...], 0.0)
    gidx = i * bn + lax.broadcasted_iota(_i32, (bn, 1), 0)
    h2 = jnp.where(gidx < _N, h2, -jnp.inf)
    bids = bid_ref[0, 0]

    @pl.when(i == 0)
    def _():
        hg_ref[...] = jnp.full((_B, _OD), -jnp.inf, _f32)

    lo = jnp.min(bids)
    hi = jnp.max(bids)

    def body(b, _):
        m = jnp.max(jnp.where(bids[:, None] == b, h2, -jnp.inf),
                    axis=0, keepdims=True)
        hg_ref[pl.ds(b, 1), :] = jnp.maximum(hg_ref[pl.ds(b, 1), :], m)
        return _

    lax.fori_loop(lo, hi + 1, body, None)


def _tcc(parts, bids3, b2c):
    bn = 1024
    return pl.pallas_call(
        _tcc_body,
        grid=(_NP // bn,),
        in_specs=[
            pl.BlockSpec((2, bn, _ODP), lambda n: (0, n, 0)),
            pl.BlockSpec((1, 1, bn), lambda n: (n, 0, 0)),
            pl.BlockSpec((1, _OD), lambda n: (0, 0)),
        ],
        out_specs=pl.BlockSpec((_B, _OD), lambda n: (0, 0)),
        out_shape=jax.ShapeDtypeStruct((_B, _OD), _f32),
    )(parts, bids3, b2c)


def _tcd1_body(enc_ref, embT_ref, w_ref, b_ref, y_ref, st_ref):
    b = pl.program_id(0)
    enc = enc_ref[0, 0]
    oh = (lax.broadcasted_iota(_i32, (26, _L), 0) == enc[None, :]).astype(_f32)
    xtT = jnp.dot(embT_ref[...], oh, preferred_element_type=_f32,
                  precision=_PREC_HI)
    t_out = _L - 16 + 1
    acc = jnp.zeros((32, t_out), _f32)
    for k in range(16):
        acc = acc + jnp.dot(w_ref[:, :, k], xtT[:, k:k + t_out],
                            preferred_element_type=_f32, precision=_PREC)
    y = acc + b_ref[...]
    y_ref[0] = y

    @pl.when(b == 0)
    def _():
        st_ref[...] = jnp.zeros_like(st_ref)

    st_ref[0:1, :] += jnp.sum(y, axis=1)[None, :]
    st_ref[1:2, :] += jnp.sum(y * y, axis=1)[None, :]


def _tcd1(enc3, embT, w1, c1b):
    t_out = _L - 15
    return pl.pallas_call(
        _tcd1_body,
        grid=(_B,),
        in_specs=[
            pl.BlockSpec((1, 1, _L), lambda b: (b, 0, 0)),
            pl.BlockSpec((128, 26), lambda b: (0, 0)),
            pl.BlockSpec((32, 128, 16), lambda b: (0, 0, 0)),
            pl.BlockSpec((32, 1), lambda b: (0, 0)),
        ],
        out_specs=[
            pl.BlockSpec((1, 32, t_out), lambda b: (b, 0, 0)),
            pl.BlockSpec((2, 32), lambda b: (0, 0)),
        ],
        out_shape=[
            jax.ShapeDtypeStruct((_B, 32, t_out), _f32),
            jax.ShapeDtypeStruct((2, 32), _f32),
        ],
    )(enc3, embT, w1, c1b)


def _bn_scale_shift(st_ref, g_ref, bb_ref, m_count):
    m = st_ref[0:1, :] * (1.0 / m_count)
    var = st_ref[1:2, :] * (1.0 / m_count) - m * m
    inv = lax.rsqrt(var + 1e-5)
    scale = g_ref[...].T * inv
    shift = bb_ref[...].T - m * scale
    return scale, shift


def _tcd23_body(cin, t_in, t_out, y_ref, st_ref, g_ref, bb_ref, w_ref, b_ref,
                yo_ref, so_ref):
    b = pl.program_id(0)
    scale, shift = _bn_scale_shift(st_ref, g_ref, bb_ref, _B * t_in)
    x = y_ref[0] * scale.T + shift.T
    x = jnp.maximum(x, 0.0)
    cout = w_ref.shape[0]
    acc = jnp.zeros((cout, t_out), _f32)
    for k in range(16):
        acc = acc + jnp.dot(w_ref[:, :, k], x[:, k:k + t_out],
                            preferred_element_type=_f32, precision=_PREC)
    y = acc + b_ref[...]
    yo_ref[0] = y

    @pl.when(b == 0)
    def _():
        so_ref[...] = jnp.zeros_like(so_ref)

    so_ref[0:1, :] += jnp.sum(y, axis=1)[None, :]
    so_ref[1:2, :] += jnp.sum(y * y, axis=1)[None, :]


def _tcd23(y, st, g, bb, w, cb, cin, cout, t_in):
    t_out = t_in - 15
    return pl.pallas_call(
        functools.partial(_tcd23_body, cin, t_in, t_out),
        grid=(_B,),
        in_specs=[
            pl.BlockSpec((1, cin, t_in), lambda b: (b, 0, 0)),
            pl.BlockSpec((2, cin), lambda b: (0, 0)),
            pl.BlockSpec((cin, 1), lambda b: (0, 0)),
            pl.BlockSpec((cin, 1), lambda b: (0, 0)),
            pl.BlockSpec((cout, cin, 16), lambda b: (0, 0, 0)),
            pl.BlockSpec((cout, 1), lambda b: (0, 0)),
        ],
        out_specs=[
            pl.BlockSpec((1, cout, t_out), lambda b: (b, 0, 0)),
            pl.BlockSpec((2, cout), lambda b: (0, 0)),
        ],
        out_shape=[
            jax.ShapeDtypeStruct((_B, cout, t_out), _f32),
            jax.ShapeDtypeStruct((2, cout), _f32),
        ],
    )(y, st, g, bb, w, cb)


def _tce_body(y_ref, st_ref, g_ref, bb_ref, w_ref, b_ref, xo_ref, so_ref):
    b = pl.program_id(0)
    t_in = y_ref.shape[2]
    scale, shift = _bn_scale_shift(st_ref, g_ref, bb_ref, _B * t_in)
    x = jnp.maximum(y_ref[0] * scale.T + shift.T, 0.0)
    pmax = jnp.max(x, axis=1)[None, :]
    xt = jnp.dot(pmax, w_ref[...], preferred_element_type=_f32,
                 precision=_PREC) + b_ref[...]
    xo_ref[0] = xt

    @pl.when(b == 0)
    def _():
        so_ref[...] = jnp.zeros_like(so_ref)

    so_ref[0:1, :] += xt
    so_ref[1:2, :] += xt * xt


def _tce(y3, st3, g, bb, w, fb):
    t_in = y3.shape[2]
    return pl.pallas_call(
        _tce_body,
        grid=(_B,),
        in_specs=[
            pl.BlockSpec((1, 96, t_in), lambda b: (b, 0, 0)),
            pl.BlockSpec((2, 96), lambda b: (0, 0)),
            pl.BlockSpec((96, 1), lambda b: (0, 0)),
            pl.BlockSpec((96, 1), lambda b: (0, 0)),
            pl.BlockSpec((96, _OD), lambda b: (0, 0)),
            pl.BlockSpec((1, _OD), lambda b: (0, 0)),
        ],
        out_specs=[
            pl.BlockSpec((1, 1, _OD), lambda b: (b, 0, 0)),
            pl.BlockSpec((2, _OD), lambda b: (0, 0)),
        ],
        out_shape=[
            jax.ShapeDtypeStruct((_B, 1, _OD), _f32),
            jax.ShapeDtypeStruct((2, _OD), _f32),
        ],
    )(y3, st3, g, bb, w, fb)


def _tcf_body(hg_ref, xt_ref, st_ref, g_ref, bb_ref, fgw_ref, fgb_ref,
              f1w_ref, f1b_ref, f2w_ref, f2b_ref, ow_ref, ob_ref, o_ref):
    hg = hg_ref[...]
    hg = jnp.where(hg > -1e30, hg, 0.0)
    xg = jnp.maximum(jnp.dot(hg, fgw_ref[...], preferred_element_type=_f32,
                             precision=_PREC) + fgb_ref[...], 0.0)
    xt = xt_ref[...].reshape(_B, _OD)
    m = st_ref[0:1, :] * (1.0 / _B)
    var = st_ref[1:2, :] * (1.0 / _B) - m * m
    inv = lax.rsqrt(var + 1e-5)
    xtn = jnp.maximum((xt - m) * inv * g_ref[...] + bb_ref[...], 0.0)
    xc = jnp.concatenate([xg, xtn], axis=1)
    y = jnp.maximum(jnp.dot(xc, f1w_ref[...], preferred_element_type=_f32,
                            precision=_PREC) + f1b_ref[...], 0.0)
    y = jnp.maximum(jnp.dot(y, f2w_ref[...], preferred_element_type=_f32,
                            precision=_PREC) + f2b_ref[...], 0.0)
    o_ref[...] = jnp.dot(y, ow_ref[...], preferred_element_type=_f32,
                         precision=_PREC) + ob_ref[...]


def _tcf(hg, xt3, stf, bnfg, bnfb, fgw, fgb, f1w, f1b, f2w, f2b, ow, ob):
    return pl.pallas_call(
        _tcf_body,
        out_shape=jax.ShapeDtypeStruct((_B, 1), _f32),
    )(hg, xt3, stf, bnfg, bnfb, fgw, fgb, f1w, f1b, f2w, f2b, ow, ob)


def kernel(x, edge_index, batch, target_encoding, W1, att_src1, att_dst1, b1,
           W2, att_src2, att_dst2, b2, fc_g1_w, fc_g1_b, emb_xt, conv1_w,
           conv1_b, bn1_g, bn1_b, conv2_w, conv2_b, bn2_g, bn2_b, conv3_w,
           conv3_b, bn3_g, bn3_b, fc_xt_w, fc_xt_b, bnf_g, bnf_b, fc1_w,
           fc1_b, fc2_w, fc2_b, out_w, out_b):
    loop = jnp.arange(_N, dtype=_i32)
    padv = jnp.arange(_EPAD - _ER, dtype=_i32) % _N
    srcp = jnp.concatenate([edge_index[0].astype(_i32), loop, padv])
    dstp = jnp.concatenate([edge_index[1].astype(_i32), loop, padv])

    w1h = jnp.pad(W1.reshape(_F, _H, _F).transpose(1, 0, 2),
                  ((0, 0), (0, 0), (0, _FP - _F)))
    asw = jnp.pad(att_src1, ((0, 0), (0, _FP - _F)))[:, None, :]
    adw = jnp.pad(att_dst1, ((0, 0), (0, _FP - _F)))[:, None, :]
    xp = jnp.pad(x, ((0, _NP - _N), (0, 0)))
    h1, as1, ad1 = _tca(xp, w1h, asw, adw)

    zeros1 = jnp.zeros((_RPT, _FP), _f32)
    out1 = _sc1(h1.reshape(_H * _NP, _FP), as1.reshape(-1), ad1.reshape(-1),
                srcp, dstp, zeros1)

    w2p = jnp.pad(W2.reshape(_H, _F, _OD),
                  ((0, 0), (0, _FP - _F), (0, 0))).reshape(_H * _FP, _OD)
    b1p = jnp.pad(b1.reshape(_H, _F),
                  ((0, 0), (0, _FP - _F))).reshape(1, _H * _FP)
    h2aug, as2, ad2 = _tcb(out1, w2p, b1p, att_src2, att_dst2)

    zeros2 = jnp.zeros((_RPT, _ODP), _f32)
    parts = _sc2(h2aug, as2.reshape(-1), ad2.reshape(-1), srcp, dstp, zeros2)

    batchp = jnp.pad(batch.astype(_i32), (0, _NP - _N), mode='edge')
    bids3 = batchp.reshape(_NP // 1024, 1, 1024)
    hg = _tcc(parts, bids3, b2.reshape(1, _OD))

    enc3 = target_encoding.astype(_i32).reshape(_B, 1, _L)
    y1, st1 = _tcd1(enc3, emb_xt.T, conv1_w, conv1_b.reshape(32, 1))
    y2, st2 = _tcd23(y1, st1, bn1_g.reshape(32, 1), bn1_b.reshape(32, 1),
                     conv2_w, conv2_b.reshape(64, 1), 32, 64, 985)
    y3, st3 = _tcd23(y2, st2, bn2_g.reshape(64, 1), bn2_b.reshape(64, 1),
                     conv3_w, conv3_b.reshape(96, 1), 64, 96, 970)
    xt3, stf = _tce(y3, st3, bn3_g.reshape(96, 1), bn3_b.reshape(96, 1),
                    fc_xt_w, fc_xt_b.reshape(1, _OD))

    return _tcf(hg, xt3, stf, bnf_g.reshape(1, _OD), bnf_b.reshape(1, _OD),
                fc_g1_w, fc_g1_b.reshape(1, _OD), fc1_w, fc1_b.reshape(1, 1024),
                fc2_w, fc2_b.reshape(1, 256), out_w, out_b.reshape(1, 1))


# speedup vs baseline: 5.9804x; 1.1997x over previous
import functools
import jax
import jax.numpy as jnp
from jax import lax
from jax.experimental import pallas as pl
from jax.experimental.pallas import tpu as pltpu
from jax.experimental.pallas import tpu_sc as plsc

_N = 10000
_NP = 10240
_E = 160000
_B = 128
_L = 1000
_F = 78
_FP = 80
_H = 10
_OD = 128
_ODP = 144
_ER = _E + _N
_EPAD = 172032
_CH = 128
_NT = 16
_RPT = _NP // 16

_f32 = jnp.float32
_i32 = jnp.int32
_PREC = lax.Precision.DEFAULT
_PREC_HI = lax.Precision.HIGHEST


def _lane_bcast(v, k):
    return lax.gather(
        v, jnp.full((16, 1), k, _i32),
        lax.GatherDimensionNumbers(offset_dims=(), collapsed_slice_dims=(0,),
                                   start_index_map=(0,)),
        (1,), mode=lax.GatherScatterMode.PROMISE_IN_BOUNDS)


def _tca_body(x_ref, w_ref, as_ref, ad_ref, h_out, aso, ado):
    xb = x_ref[...]
    h = jnp.dot(xb, w_ref[0], preferred_element_type=_f32, precision=_PREC)
    lane = lax.broadcasted_iota(_i32, h.shape, 1)
    h = jnp.where(lane == _F, 1.0, h)
    h_out[0] = h
    aso[0, 0] = jnp.sum(h * as_ref[0], axis=1)
    ado[0, 0] = jnp.sum(h * ad_ref[0], axis=1)


def _tca(x, w1h, asw, adw):
    bn = 1024
    return pl.pallas_call(
        _tca_body,
        grid=(_H, _NP // bn),
        in_specs=[
            pl.BlockSpec((bn, _F), lambda h, n: (n, 0)),
            pl.BlockSpec((1, _F, _FP), lambda h, n: (h, 0, 0)),
            pl.BlockSpec((1, 1, _FP), lambda h, n: (h, 0, 0)),
            pl.BlockSpec((1, 1, _FP), lambda h, n: (h, 0, 0)),
        ],
        out_specs=[
            pl.BlockSpec((1, bn, _FP), lambda h, n: (h, n, 0)),
            pl.BlockSpec((1, 1, bn), lambda h, n: (h, 0, n)),
            pl.BlockSpec((1, 1, bn), lambda h, n: (h, 0, n)),
        ],
        out_shape=[
            jax.ShapeDtypeStruct((_H, _NP, _FP), _f32),
            jax.ShapeDtypeStruct((_H, 1, _NP), _f32),
            jax.ShapeDtypeStruct((_H, 1, _NP), _f32),
        ],
    )(x, w1h, asw, adw)


def _sc1_body(h_hbm, as_hbm, ad_hbm, src_hbm, dst_hbm, zero_hbm, out_hbm,
              acc, idx_s, idx_g, idx_g2, idx_d0, idx_d1, rows0, rows1,
              asb, adb, sem0, sem1, sem2, sem3, sem4, sem_sc0, sem_sc1):
    c = lax.axis_index("c")
    s = lax.axis_index("s")
    npair = _EPAD // _NT // _CH // 2
    idx_d = (idx_d0, idx_d1)
    rows = (rows0, rows1)
    sem_sc = (sem_sc0, sem_sc1)

    def head_body(hi, _):
        head = hi * 2 + c
        pltpu.sync_copy(zero_hbm, acc.at[pl.ds(s * _RPT, _RPT)])
        plsc.subcore_barrier()

        def pair_body(i2, _):
            for b in range(2):
                ci = i2 * 2 + b
                off = s * (_EPAD // _NT) + ci * _CH

                @pl.when(i2 > 0)
                def _():
                    pltpu.make_async_copy(rows[b], acc.at[idx_d[b]],
                                          sem_sc[b]).wait()

                cp_s = pltpu.async_copy(src_hbm.at[pl.ds(off, _CH)], idx_s, sem0)
                cp_d = pltpu.async_copy(dst_hbm.at[pl.ds(off, _CH)], idx_d[b], sem1)
                cp_s.wait()
                hoff = head * _NP
                for j in range(_CH // 16):
                    sl = pl.ds(j * 16, 16)
                    idx_g[sl] = idx_s[sl] + hoff
                cp_r = pltpu.async_copy(h_hbm.at[idx_g], rows[b], sem2)
                cp_a = pltpu.async_copy(as_hbm.at[idx_g], asb, sem3)
                cp_d.wait()
                for j in range(_CH // 16):
                    sl = pl.ds(j * 16, 16)
                    idx_g2[sl] = idx_d[b][sl] + hoff
                cp_ad = pltpu.async_copy(ad_hbm.at[idx_g2], adb, sem4)
                cp_a.wait()
                cp_ad.wait()
                cp_r.wait()
                for g in range(_CH // 16):
                    sl = pl.ds(g * 16, 16)
                    e = asb[sl] + adb[sl]
                    e = jnp.maximum(e, 0.2 * e)
                    w = jnp.exp(e)
                    ei = off + g * 16 + lax.iota(_i32, 16)
                    wg = jnp.where(ei < _ER, w, 0.0)
                    for k in range(16):
                        wv = _lane_bcast(wg, k)
                        kk = g * 16 + k
                        for q in range(_FP // 16):
                            sl2 = pl.ds(q * 16, 16)
                            rows[b][kk, sl2] = rows[b][kk, sl2] * wv
                pltpu.async_copy(rows[b], acc.at[idx_d[b]], sem_sc[b], add=True)
            return _

        lax.fori_loop(0, npair, pair_body, None)
        pltpu.make_async_copy(rows[0], acc.at[idx_d[0]], sem_sc[0]).wait()
        pltpu.make_async_copy(rows[1], acc.at[idx_d[1]], sem_sc[1]).wait()
        plsc.subcore_barrier()
        pltpu.sync_copy(acc.at[pl.ds(s * _RPT, _RPT)],
                        out_hbm.at[pl.ds(s * _RPT, _RPT), head])
        plsc.subcore_barrier()
        return _

    lax.fori_loop(0, _H // 2, head_body, None)


def _sc1(h_flat, as_flat, ad_flat, srcp, dstp, zeros1):
    mesh = plsc.VectorSubcoreMesh(core_axis_name="c", subcore_axis_name="s")
    return pl.kernel(
        _sc1_body,
        out_type=jax.ShapeDtypeStruct((_NP, _H, _FP), _f32),
        mesh=mesh,
        compiler_params=pltpu.CompilerParams(use_tc_tiling_on_sc=False),
        scratch_types=[
            pltpu.VMEM_SHARED((_NP, _FP), _f32),
            pltpu.VMEM((_CH,), _i32),
            pltpu.VMEM((_CH,), _i32),
            pltpu.VMEM((_CH,), _i32),
            pltpu.VMEM((_CH,), _i32),
            pltpu.VMEM((_CH,), _i32),
            pltpu.VMEM((_CH, _FP), _f32),
            pltpu.VMEM((_CH, _FP), _f32),
            pltpu.VMEM((_CH,), _f32),
            pltpu.VMEM((_CH,), _f32),
            pltpu.SemaphoreType.DMA,
            pltpu.SemaphoreType.DMA,
            pltpu.SemaphoreType.DMA,
            pltpu.SemaphoreType.DMA,
            pltpu.SemaphoreType.DMA,
            pltpu.SemaphoreType.DMA,
            pltpu.SemaphoreType.DMA,
        ],
    )(h_flat, as_flat, ad_flat, srcp, dstp, zeros1)


def _tcb_body(v_ref, w2_ref, b1_ref, a2s_ref, a2d_ref, h2o, aso, ado):
    bn = v_ref.shape[0]
    v = v_ref[...]
    den = v[:, :, _F:_F + 1] + 1e-16
    v = (v / den).reshape(bn, _H * _FP)
    v = v + b1_ref[0]
    x2 = jnp.where(v > 0, v, jnp.exp(jnp.minimum(v, 0.0)) - 1.0)
    h2 = jnp.dot(x2, w2_ref[...], preferred_element_type=_f32, precision=_PREC)
    ones = jnp.ones((bn, 1), _f32)
    zer = jnp.zeros((bn, _ODP - _OD - 1), _f32)
    h2o[...] = jnp.concatenate([h2, ones, zer], axis=1)
    aso[0] = jnp.sum(h2 * a2s_ref[...], axis=1)
    ado[0] = jnp.sum(h2 * a2d_ref[...], axis=1)


def _tcb(out1, w2p, b1p, a2s, a2d):
    bn = 1024
    return pl.pallas_call(
        _tcb_body,
        grid=(_NP // bn,),
        in_specs=[
            pl.BlockSpec((bn, _H, _FP), lambda n: (n, 0, 0)),
            pl.BlockSpec((_H * _FP, _OD), lambda n: (0, 0)),
            pl.BlockSpec((1, _H * _FP), lambda n: (0, 0)),
            pl.BlockSpec((1, _OD), lambda n: (0, 0)),
            pl.BlockSpec((1, _OD), lambda n: (0, 0)),
        ],
        out_specs=[
            pl.BlockSpec((bn, _ODP), lambda n: (n, 0)),
            pl.BlockSpec((1, bn), lambda n: (0, n)),
            pl.BlockSpec((1, bn), lambda n: (0, n)),
        ],
        out_shape=[
            jax.ShapeDtypeStruct((_NP, _ODP), _f32),
            jax.ShapeDtypeStruct((1, _NP), _f32),
            jax.ShapeDtypeStruct((1, _NP), _f32),
        ],
    )(out1, w2p, b1p, a2s, a2d)


def _sc2_body(h_hbm, as_hbm, ad_hbm, src_hbm, dst_hbm, zero_hbm, out_hbm,
              acc, idx_s, idx_d0, idx_d1, rows0, rows1, asb, adb,
              sem0, sem1, sem2, sem3, sem4, sem_sc0, sem_sc1):
    c = lax.axis_index("c")
    s = lax.axis_index("s")
    per_tile = _EPAD // 32
    npair = per_tile // _CH // 2
    idx_d = (idx_d0, idx_d1)
    rows = (rows0, rows1)
    sem_sc = (sem_sc0, sem_sc1)

    pltpu.sync_copy(zero_hbm, acc.at[pl.ds(s * _RPT, _RPT)])
    plsc.subcore_barrier()

    def pair_body(i2, _):
        for b in range(2):
            ci = i2 * 2 + b
            off = (c * _NT + s) * per_tile + ci * _CH

            @pl.when(i2 > 0)
            def _():
                pltpu.make_async_copy(rows[b], acc.at[idx_d[b]],
                                      sem_sc[b]).wait()

            cp_s = pltpu.async_copy(src_hbm.at[pl.ds(off, _CH)], idx_s, sem0)
            cp_d = pltpu.async_copy(dst_hbm.at[pl.ds(off, _CH)], idx_d[b], sem1)
            cp_s.wait()
            cp_r = pltpu.async_copy(h_hbm.at[idx_s], rows[b], sem2)
            cp_a = pltpu.async_copy(as_hbm.at[idx_s], asb, sem3)
            cp_d.wait()
            cp_ad = pltpu.async_copy(ad_hbm.at[idx_d[b]], adb, sem4)
            cp_a.wait()
            cp_ad.wait()
            cp_r.wait()
            for g in range(_CH // 16):
                sl = pl.ds(g * 16, 16)
                e = asb[sl] + adb[sl]
                e = jnp.maximum(e, 0.2 * e)
                w = jnp.exp(e)
                ei = off + g * 16 + lax.iota(_i32, 16)
                wg = jnp.where(ei < _ER, w, 0.0)
                for k in range(16):
                    wv = _lane_bcast(wg, k)
                    kk = g * 16 + k
                    for q in range(_ODP // 16):
                        sl2 = pl.ds(q * 16, 16)
                        rows[b][kk, sl2] = rows[b][kk, sl2] * wv
            pltpu.async_copy(rows[b], acc.at[idx_d[b]], sem_sc[b], add=True)
        return _

    lax.fori_loop(0, npair, pair_body, None)
    pltpu.make_async_copy(rows[0], acc.at[idx_d[0]], sem_sc[0]).wait()
    pltpu.make_async_copy(rows[1], acc.at[idx_d[1]], sem_sc[1]).wait()
    plsc.subcore_barrier()
    pltpu.sync_copy(acc.at[pl.ds(s * _RPT, _RPT)],
                    out_hbm.at[c, pl.ds(s * _RPT, _RPT)])


def _sc2(h2aug, as2, ad2, srcp, dstp, zeros2):
    mesh = plsc.VectorSubcoreMesh(core_axis_name="c", subcore_axis_name="s")
    return pl.kernel(
        _sc2_body,
        out_type=jax.ShapeDtypeStruct((2, _NP, _ODP), _f32),
        mesh=mesh,
        compiler_params=pltpu.CompilerParams(use_tc_tiling_on_sc=False),
        scratch_types=[
            pltpu.VMEM_SHARED((_NP, _ODP), _f32),
            pltpu.VMEM((_CH,), _i32),
            pltpu.VMEM((_CH,), _i32),
            pltpu.VMEM((_CH,), _i32),
            pltpu.VMEM((_CH, _ODP), _f32),
            pltpu.VMEM((_CH, _ODP), _f32),
            pltpu.VMEM((_CH,), _f32),
            pltpu.VMEM((_CH,), _f32),
            pltpu.SemaphoreType.DMA,
            pltpu.SemaphoreType.DMA,
            pltpu.SemaphoreType.DMA,
            pltpu.SemaphoreType.DMA,
            pltpu.SemaphoreType.DMA,
            pltpu.SemaphoreType.DMA,
            pltpu.SemaphoreType.DMA,
        ],
    )(h2aug, as2, ad2, srcp, dstp, zeros2)


def _tcc_body(p_ref, bid_ref, b2_ref, hg_ref):
    i = pl.program_id(0)
    bn = p_ref.shape[1]
    p = p_ref[...]
    num = p[0, :, :_OD] + p[1, :, :_OD]
    den = p[0, :, _OD:_OD + 1] + p[1, :, _OD:_OD + 1] + 1e-16
    h2 = jnp.maximum(num / den + b2_ref[...], 0.0)
    gidx = i * bn + lax.broadcasted_iota(_i32, (bn, 1), 0)
    h2 = jnp.where(gidx < _N, h2, -jnp.inf)
    bids = bid_ref[0, 0]

    @pl.when(i == 0)
    def _():
        hg_ref[...] = jnp.full((_B, _OD), -jnp.inf, _f32)

    lo = jnp.min(bids)
    hi = jnp.max(bids)

    def body(b, _):
        m = jnp.max(jnp.where(bids[:, None] == b, h2, -jnp.inf),
                    axis=0, keepdims=True)
        hg_ref[pl.ds(b, 1), :] = jnp.maximum(hg_ref[pl.ds(b, 1), :], m)
        return _

    lax.fori_loop(lo, hi + 1, body, None)


def _tcc(parts, bids3, b2c):
    bn = 1024
    return pl.pallas_call(
        _tcc_body,
        grid=(_NP // bn,),
        in_specs=[
            pl.BlockSpec((2, bn, _ODP), lambda n: (0, n, 0)),
            pl.BlockSpec((1, 1, bn), lambda n: (n, 0, 0)),
            pl.BlockSpec((1, _OD), lambda n: (0, 0)),
        ],
        out_specs=pl.BlockSpec((_B, _OD), lambda n: (0, 0)),
        out_shape=jax.ShapeDtypeStruct((_B, _OD), _f32),
    )(parts, bids3, b2c)


def _tcd1_body(enc_ref, embT_ref, w_ref, b_ref, y_ref, st_ref):
    b = pl.program_id(0)
    enc = enc_ref[0, 0]
    oh = (lax.broadcasted_iota(_i32, (26, _L), 0) == enc[None, :]).astype(_f32)
    xtT = jnp.dot(embT_ref[...], oh, preferred_element_type=_f32,
                  precision=_PREC_HI)
    t_out = _L - 16 + 1
    acc = jnp.zeros((32, t_out), _f32)
    for k in range(16):
        acc = acc + jnp.dot(w_ref[:, :, k], xtT[:, k:k + t_out],
                            preferred_element_type=_f32, precision=_PREC)
    y = acc + b_ref[...]
    y_ref[0] = y

    @pl.when(b == 0)
    def _():
        st_ref[...] = jnp.zeros_like(st_ref)

    st_ref[0:1, :] += jnp.sum(y, axis=1)[None, :]
    st_ref[1:2, :] += jnp.sum(y * y, axis=1)[None, :]


def _tcd1(enc3, embT, w1, c1b):
    t_out = _L - 15
    return pl.pallas_call(
        _tcd1_body,
        grid=(_B,),
        in_specs=[
            pl.BlockSpec((1, 1, _L), lambda b: (b, 0, 0)),
            pl.BlockSpec((128, 26), lambda b: (0, 0)),
            pl.BlockSpec((32, 128, 16), lambda b: (0, 0, 0)),
            pl.BlockSpec((32, 1), lambda b: (0, 0)),
        ],
        out_specs=[
            pl.BlockSpec((1, 32, t_out), lambda b: (b, 0, 0)),
            pl.BlockSpec((2, 32), lambda b: (0, 0)),
        ],
        out_shape=[
            jax.ShapeDtypeStruct((_B, 32, t_out), _f32),
            jax.ShapeDtypeStruct((2, 32), _f32),
        ],
    )(enc3, embT, w1, c1b)


def _bn_scale_shift(st_ref, g_ref, bb_ref, m_count):
    m = st_ref[0:1, :] * (1.0 / m_count)
    var = st_ref[1:2, :] * (1.0 / m_count) - m * m
    inv = lax.rsqrt(var + 1e-5)
    scale = g_ref[...].T * inv
    shift = bb_ref[...].T - m * scale
    return scale, shift


def _tcd23_body(cin, t_in, t_out, y_ref, st_ref, g_ref, bb_ref, w_ref, b_ref,
                yo_ref, so_ref):
    b = pl.program_id(0)
    scale, shift = _bn_scale_shift(st_ref, g_ref, bb_ref, _B * t_in)
    x = y_ref[0] * scale.T + shift.T
    x = jnp.maximum(x, 0.0)
    cout = w_ref.shape[0]
    acc = jnp.zeros((cout, t_out), _f32)
    for k in range(16):
        acc = acc + jnp.dot(w_ref[:, :, k], x[:, k:k + t_out],
                            preferred_element_type=_f32, precision=_PREC)
    y = acc + b_ref[...]
    yo_ref[0] = y

    @pl.when(b == 0)
    def _():
        so_ref[...] = jnp.zeros_like(so_ref)

    so_ref[0:1, :] += jnp.sum(y, axis=1)[None, :]
    so_ref[1:2, :] += jnp.sum(y * y, axis=1)[None, :]


def _tcd23(y, st, g, bb, w, cb, cin, cout, t_in):
    t_out = t_in - 15
    return pl.pallas_call(
        functools.partial(_tcd23_body, cin, t_in, t_out),
        grid=(_B,),
        in_specs=[
            pl.BlockSpec((1, cin, t_in), lambda b: (b, 0, 0)),
            pl.BlockSpec((2, cin), lambda b: (0, 0)),
            pl.BlockSpec((cin, 1), lambda b: (0, 0)),
            pl.BlockSpec((cin, 1), lambda b: (0, 0)),
            pl.BlockSpec((cout, cin, 16), lambda b: (0, 0, 0)),
            pl.BlockSpec((cout, 1), lambda b: (0, 0)),
        ],
        out_specs=[
            pl.BlockSpec((1, cout, t_out), lambda b: (b, 0, 0)),
            pl.BlockSpec((2, cout), lambda b: (0, 0)),
        ],
        out_shape=[
            jax.ShapeDtypeStruct((_B, cout, t_out), _f32),
            jax.ShapeDtypeStruct((2, cout), _f32),
        ],
    )(y, st, g, bb, w, cb)


def _tce_body(y_ref, st_ref, g_ref, bb_ref, w_ref, b_ref, xo_ref, so_ref):
    b = pl.program_id(0)
    t_in = y_ref.shape[2]
    scale, shift = _bn_scale_shift(st_ref, g_ref, bb_ref, _B * t_in)
    x = jnp.maximum(y_ref[0] * scale.T + shift.T, 0.0)
    pmax = jnp.max(x, axis=1)[None, :]
    xt = jnp.dot(pmax, w_ref[...], preferred_element_type=_f32,
                 precision=_PREC) + b_ref[...]
    xo_ref[0] = xt

    @pl.when(b == 0)
    def _():
        so_ref[...] = jnp.zeros_like(so_ref)

    so_ref[0:1, :] += xt
    so_ref[1:2, :] += xt * xt


def _tce(y3, st3, g, bb, w, fb):
    t_in = y3.shape[2]
    return pl.pallas_call(
        _tce_body,
        grid=(_B,),
        in_specs=[
            pl.BlockSpec((1, 96, t_in), lambda b: (b, 0, 0)),
            pl.BlockSpec((2, 96), lambda b: (0, 0)),
            pl.BlockSpec((96, 1), lambda b: (0, 0)),
            pl.BlockSpec((96, 1), lambda b: (0, 0)),
            pl.BlockSpec((96, _OD), lambda b: (0, 0)),
            pl.BlockSpec((1, _OD), lambda b: (0, 0)),
        ],
        out_specs=[
            pl.BlockSpec((1, 1, _OD), lambda b: (b, 0, 0)),
            pl.BlockSpec((2, _OD), lambda b: (0, 0)),
        ],
        out_shape=[
            jax.ShapeDtypeStruct((_B, 1, _OD), _f32),
            jax.ShapeDtypeStruct((2, _OD), _f32),
        ],
    )(y3, st3, g, bb, w, fb)


def _tcf_body(hg_ref, xt_ref, st_ref, g_ref, bb_ref, fgw_ref, fgb_ref,
              f1w_ref, f1b_ref, f2w_ref, f2b_ref, ow_ref, ob_ref, o_ref):
    hg = hg_ref[...]
    hg = jnp.where(hg > -1e30, hg, 0.0)
    xg = jnp.maximum(jnp.dot(hg, fgw_ref[...], preferred_element_type=_f32,
                             precision=_PREC) + fgb_ref[...], 0.0)
    xt = xt_ref[...].reshape(_B, _OD)
    m = st_ref[0:1, :] * (1.0 / _B)
    var = st_ref[1:2, :] * (1.0 / _B) - m * m
    inv = lax.rsqrt(var + 1e-5)
    xtn = jnp.maximum((xt - m) * inv * g_ref[...] + bb_ref[...], 0.0)
    xc = jnp.concatenate([xg, xtn], axis=1)
    y = jnp.maximum(jnp.dot(xc, f1w_ref[...], preferred_element_type=_f32,
                            precision=_PREC) + f1b_ref[...], 0.0)
    y = jnp.maximum(jnp.dot(y, f2w_ref[...], preferred_element_type=_f32,
                            precision=_PREC) + f2b_ref[...], 0.0)
    o_ref[...] = jnp.dot(y, ow_ref[...], preferred_element_type=_f32,
                         precision=_PREC) + ob_ref[...]


def _tcf(hg, xt3, stf, bnfg, bnfb, fgw, fgb, f1w, f1b, f2w, f2b, ow, ob):
    return pl.pallas_call(
        _tcf_body,
        out_shape=jax.ShapeDtypeStruct((_B, 1), _f32),
    )(hg, xt3, stf, bnfg, bnfb, fgw, fgb, f1w, f1b, f2w, f2b, ow, ob)


def kernel(x, edge_index, batch, target_encoding, W1, att_src1, att_dst1, b1,
           W2, att_src2, att_dst2, b2, fc_g1_w, fc_g1_b, emb_xt, conv1_w,
           conv1_b, bn1_g, bn1_b, conv2_w, conv2_b, bn2_g, bn2_b, conv3_w,
           conv3_b, bn3_g, bn3_b, fc_xt_w, fc_xt_b, bnf_g, bnf_b, fc1_w,
           fc1_b, fc2_w, fc2_b, out_w, out_b):
    loop = jnp.arange(_N, dtype=_i32)
    padv = jnp.arange(_EPAD - _ER, dtype=_i32) % _N
    srcp = jnp.concatenate([edge_index[0].astype(_i32), loop, padv])
    dstp = jnp.concatenate([edge_index[1].astype(_i32), loop, padv])

    w1h = jnp.pad(W1.reshape(_F, _H, _F).transpose(1, 0, 2),
                  ((0, 0), (0, 0), (0, _FP - _F)))
    asw = jnp.pad(att_src1, ((0, 0), (0, _FP - _F)))[:, None, :]
    adw = jnp.pad(att_dst1, ((0, 0), (0, _FP - _F)))[:, None, :]
    xp = jnp.pad(x, ((0, _NP - _N), (0, 0)))
    h1, as1, ad1 = _tca(xp, w1h, asw, adw)

    zeros1 = jnp.zeros((_RPT, _FP), _f32)
    out1 = _sc1(h1.reshape(_H * _NP, _FP), as1.reshape(-1), ad1.reshape(-1),
                srcp, dstp, zeros1)

    w2p = jnp.pad(W2.reshape(_H, _F, _OD),
                  ((0, 0), (0, _FP - _F), (0, 0))).reshape(_H * _FP, _OD)
    b1p = jnp.pad(b1.reshape(_H, _F),
                  ((0, 0), (0, _FP - _F))).reshape(1, _H * _FP)
    h2aug, as2, ad2 = _tcb(out1, w2p, b1p, att_src2, att_dst2)

    zeros2 = jnp.zeros((_RPT, _ODP), _f32)
    parts = _sc2(h2aug, as2.reshape(-1), ad2.reshape(-1), srcp, dstp, zeros2)

    batchp = jnp.pad(batch.astype(_i32), (0, _NP - _N), mode='edge')
    bids3 = batchp.reshape(_NP // 1024, 1, 1024)
    hg = _tcc(parts, bids3, b2.reshape(1, _OD))

    enc3 = target_encoding.astype(_i32).reshape(_B, 1, _L)
    y1, st1 = _tcd1(enc3, emb_xt.T, conv1_w, conv1_b.reshape(32, 1))
    y2, st2 = _tcd23(y1, st1, bn1_g.reshape(32, 1), bn1_b.reshape(32, 1),
                     conv2_w, conv2_b.reshape(64, 1), 32, 64, 985)
    y3, st3 = _tcd23(y2, st2, bn2_g.reshape(64, 1), bn2_b.reshape(64, 1),
                     conv3_w, conv3_b.reshape(96, 1), 64, 96, 970)
    xt3, stf = _tce(y3, st3, bn3_g.reshape(96, 1), bn3_b.reshape(96, 1),
                    fc_xt_w, fc_xt_b.reshape(1, _OD))

    return _tcf(hg, xt3, stf, bnf_g.reshape(1, _OD), bnf_b.reshape(1, _OD),
                fc_g1_w, fc_g1_b.reshape(1, _OD), fc1_w, fc1_b.reshape(1, 1024),
                fc2_w, fc2_b.reshape(1, 256), out_w, out_b.reshape(1, 1))


# speedup vs baseline: 11.2824x; 1.8866x over previous
import functools
import jax
import jax.numpy as jnp
from jax import lax
from jax.experimental import pallas as pl
from jax.experimental.pallas import tpu as pltpu
from jax.experimental.pallas import tpu_sc as plsc

_N = 10000
_NP = 10240
_E = 160000
_B = 128
_L = 1000
_F = 78
_FP = 80
_H = 10
_OD = 128
_ODP = 144
_ER = _E + _N
_EPAD = 172032
_CH = 128
_NT = 16
_RPT = _NP // 16

_f32 = jnp.float32
_i32 = jnp.int32
_PREC = lax.Precision.DEFAULT
_PREC_HI = lax.Precision.HIGHEST


def _lane_bcast(v, k):
    return lax.gather(
        v, jnp.full((16, 1), k, _i32),
        lax.GatherDimensionNumbers(offset_dims=(), collapsed_slice_dims=(0,),
                                   start_index_map=(0,)),
        (1,), mode=lax.GatherScatterMode.PROMISE_IN_BOUNDS)


def _tca_body(x_ref, w_ref, as_ref, ad_ref, h_out, aso, ado):
    xb = x_ref[...]
    h = jnp.dot(xb, w_ref[0], preferred_element_type=_f32, precision=_PREC)
    lane = lax.broadcasted_iota(_i32, h.shape, 1)
    h = jnp.where(lane == _F, 1.0, h)
    h_out[0] = h
    aso[0, 0] = jnp.sum(h * as_ref[0], axis=1)
    ado[0, 0] = jnp.sum(h * ad_ref[0], axis=1)


def _tca(x, w1h, asw, adw):
    bn = 1024
    return pl.pallas_call(
        _tca_body,
        grid=(_H, _NP // bn),
        in_specs=[
            pl.BlockSpec((bn, _F), lambda h, n: (n, 0)),
            pl.BlockSpec((1, _F, _FP), lambda h, n: (h, 0, 0)),
            pl.BlockSpec((1, 1, _FP), lambda h, n: (h, 0, 0)),
            pl.BlockSpec((1, 1, _FP), lambda h, n: (h, 0, 0)),
        ],
        out_specs=[
            pl.BlockSpec((1, bn, _FP), lambda h, n: (h, n, 0)),
            pl.BlockSpec((1, 1, bn), lambda h, n: (h, 0, n)),
            pl.BlockSpec((1, 1, bn), lambda h, n: (h, 0, n)),
        ],
        out_shape=[
            jax.ShapeDtypeStruct((_H, _NP, _FP), _f32),
            jax.ShapeDtypeStruct((_H, 1, _NP), _f32),
            jax.ShapeDtypeStruct((_H, 1, _NP), _f32),
        ],
    )(x, w1h, asw, adw)


def _sc1_body(h_hbm, as_hbm, ad_hbm, src_hbm, dst_hbm, zero_hbm, out_hbm,
              acc, idx_s, idx_g, idx_g2, idx_d0, idx_d1, rows0, rows1,
              asb, adb, sem0, sem1, sem2, sem3, sem4, sem_sc0, sem_sc1):
    c = lax.axis_index("c")
    s = lax.axis_index("s")
    npair = _EPAD // _NT // _CH // 2
    idx_d = (idx_d0, idx_d1)
    rows = (rows0, rows1)
    sem_sc = (sem_sc0, sem_sc1)

    def head_body(hi, _):
        head = hi * 2 + c
        pltpu.sync_copy(zero_hbm, acc.at[pl.ds(s * _RPT, _RPT)])
        plsc.subcore_barrier()

        def pair_body(i2, _):
            for b in range(2):
                ci = i2 * 2 + b
                off = s * (_EPAD // _NT) + ci * _CH

                @pl.when(i2 > 0)
                def _():
                    pltpu.make_async_copy(rows[b], acc.at[idx_d[b]],
                                          sem_sc[b]).wait()

                cp_s = pltpu.async_copy(src_hbm.at[pl.ds(off, _CH)], idx_s, sem0)
                cp_d = pltpu.async_copy(dst_hbm.at[pl.ds(off, _CH)], idx_d[b], sem1)
                cp_s.wait()
                hoff = head * _NP
                for j in range(_CH // 16):
                    sl = pl.ds(j * 16, 16)
                    idx_g[sl] = idx_s[sl] + hoff
                cp_r = pltpu.async_copy(h_hbm.at[idx_g], rows[b], sem2)
                cp_a = pltpu.async_copy(as_hbm.at[idx_g], asb, sem3)
                cp_d.wait()
                for j in range(_CH // 16):
                    sl = pl.ds(j * 16, 16)
                    idx_g2[sl] = idx_d[b][sl] + hoff
                cp_ad = pltpu.async_copy(ad_hbm.at[idx_g2], adb, sem4)
                cp_a.wait()
                cp_ad.wait()
                cp_r.wait()
                for g in range(_CH // 16):
                    sl = pl.ds(g * 16, 16)
                    e = asb[sl] + adb[sl]
                    e = jnp.maximum(e, 0.2 * e)
                    w = jnp.exp(e)
                    ei = off + g * 16 + lax.iota(_i32, 16)
                    wg = jnp.where(ei < _ER, w, 0.0)
                    for k in range(16):
                        wv = _lane_bcast(wg, k)
                        kk = g * 16 + k
                        for q in range(_FP // 16):
                            sl2 = pl.ds(q * 16, 16)
                            rows[b][kk, sl2] = rows[b][kk, sl2] * wv
                pltpu.async_copy(rows[b], acc.at[idx_d[b]], sem_sc[b], add=True)
            return _

        lax.fori_loop(0, npair, pair_body, None)
        pltpu.make_async_copy(rows[0], acc.at[idx_d[0]], sem_sc[0]).wait()
        pltpu.make_async_copy(rows[1], acc.at[idx_d[1]], sem_sc[1]).wait()
        plsc.subcore_barrier()
        pltpu.sync_copy(acc.at[pl.ds(s * _RPT, _RPT)],
                        out_hbm.at[pl.ds(s * _RPT, _RPT), head])
        plsc.subcore_barrier()
        return _

    lax.fori_loop(0, _H // 2, head_body, None)


def _sc1(h_flat, as_flat, ad_flat, srcp, dstp, zeros1):
    mesh = plsc.VectorSubcoreMesh(core_axis_name="c", subcore_axis_name="s")
    return pl.kernel(
        _sc1_body,
        out_type=jax.ShapeDtypeStruct((_NP, _H, _FP), _f32),
        mesh=mesh,
        compiler_params=pltpu.CompilerParams(use_tc_tiling_on_sc=False),
        scratch_types=[
            pltpu.VMEM_SHARED((_NP, _FP), _f32),
            pltpu.VMEM((_CH,), _i32),
            pltpu.VMEM((_CH,), _i32),
            pltpu.VMEM((_CH,), _i32),
            pltpu.VMEM((_CH,), _i32),
            pltpu.VMEM((_CH,), _i32),
            pltpu.VMEM((_CH, _FP), _f32),
            pltpu.VMEM((_CH, _FP), _f32),
            pltpu.VMEM((_CH,), _f32),
            pltpu.VMEM((_CH,), _f32),
            pltpu.SemaphoreType.DMA,
            pltpu.SemaphoreType.DMA,
            pltpu.SemaphoreType.DMA,
            pltpu.SemaphoreType.DMA,
            pltpu.SemaphoreType.DMA,
            pltpu.SemaphoreType.DMA,
            pltpu.SemaphoreType.DMA,
        ],
    )(h_flat, as_flat, ad_flat, srcp, dstp, zeros1)


def _tcb_body(v_ref, w2_ref, b1_ref, a2s_ref, a2d_ref, h2o, aso, ado):
    bn = v_ref.shape[0]
    v = v_ref[...]
    den = v[:, :, _F:_F + 1] + 1e-16
    v = (v / den).reshape(bn, _H * _FP)
    v = v + b1_ref[0]
    x2 = jnp.where(v > 0, v, jnp.exp(jnp.minimum(v, 0.0)) - 1.0)
    h2 = jnp.dot(x2, w2_ref[...], preferred_element_type=_f32, precision=_PREC)
    ones = jnp.ones((bn, 1), _f32)
    zer = jnp.zeros((bn, _ODP - _OD - 1), _f32)
    h2o[...] = jnp.concatenate([h2, ones, zer], axis=1)
    aso[0] = jnp.sum(h2 * a2s_ref[...], axis=1)
    ado[0] = jnp.sum(h2 * a2d_ref[...], axis=1)


def _tcb(out1, w2p, b1p, a2s, a2d):
    bn = 1024
    return pl.pallas_call(
        _tcb_body,
        grid=(_NP // bn,),
        in_specs=[
            pl.BlockSpec((bn, _H, _FP), lambda n: (n, 0, 0)),
            pl.BlockSpec((_H * _FP, _OD), lambda n: (0, 0)),
            pl.BlockSpec((1, _H * _FP), lambda n: (0, 0)),
            pl.BlockSpec((1, _OD), lambda n: (0, 0)),
            pl.BlockSpec((1, _OD), lambda n: (0, 0)),
        ],
        out_specs=[
            pl.BlockSpec((bn, _ODP), lambda n: (n, 0)),
            pl.BlockSpec((1, bn), lambda n: (0, n)),
            pl.BlockSpec((1, bn), lambda n: (0, n)),
        ],
        out_shape=[
            jax.ShapeDtypeStruct((_NP, _ODP), _f32),
            jax.ShapeDtypeStruct((1, _NP), _f32),
            jax.ShapeDtypeStruct((1, _NP), _f32),
        ],
    )(out1, w2p, b1p, a2s, a2d)


def _sc2_body(h_hbm, as_hbm, ad_hbm, src_hbm, dst_hbm, zero_hbm, out_hbm,
              acc, idx_s, idx_d0, idx_d1, rows0, rows1, asb, adb,
              sem0, sem1, sem2, sem3, sem4, sem_sc0, sem_sc1):
    c = lax.axis_index("c")
    s = lax.axis_index("s")
    per_tile = _EPAD // 32
    npair = per_tile // _CH // 2
    idx_d = (idx_d0, idx_d1)
    rows = (rows0, rows1)
    sem_sc = (sem_sc0, sem_sc1)

    pltpu.sync_copy(zero_hbm, acc.at[pl.ds(s * _RPT, _RPT)])
    plsc.subcore_barrier()

    def pair_body(i2, _):
        for b in range(2):
            ci = i2 * 2 + b
            off = (c * _NT + s) * per_tile + ci * _CH

            @pl.when(i2 > 0)
            def _():
                pltpu.make_async_copy(rows[b], acc.at[idx_d[b]],
                                      sem_sc[b]).wait()

            cp_s = pltpu.async_copy(src_hbm.at[pl.ds(off, _CH)], idx_s, sem0)
            cp_d = pltpu.async_copy(dst_hbm.at[pl.ds(off, _CH)], idx_d[b], sem1)
            cp_s.wait()
            cp_r = pltpu.async_copy(h_hbm.at[idx_s], rows[b], sem2)
            cp_a = pltpu.async_copy(as_hbm.at[idx_s], asb, sem3)
            cp_d.wait()
            cp_ad = pltpu.async_copy(ad_hbm.at[idx_d[b]], adb, sem4)
            cp_a.wait()
            cp_ad.wait()
            cp_r.wait()
            for g in range(_CH // 16):
                sl = pl.ds(g * 16, 16)
                e = asb[sl] + adb[sl]
                e = jnp.maximum(e, 0.2 * e)
                w = jnp.exp(e)
                ei = off + g * 16 + lax.iota(_i32, 16)
                wg = jnp.where(ei < _ER, w, 0.0)
                for k in range(16):
                    wv = _lane_bcast(wg, k)
                    kk = g * 16 + k
                    for q in range(_ODP // 16):
                        sl2 = pl.ds(q * 16, 16)
                        rows[b][kk, sl2] = rows[b][kk, sl2] * wv
            pltpu.async_copy(rows[b], acc.at[idx_d[b]], sem_sc[b], add=True)
        return _

    lax.fori_loop(0, npair, pair_body, None)
    pltpu.make_async_copy(rows[0], acc.at[idx_d[0]], sem_sc[0]).wait()
    pltpu.make_async_copy(rows[1], acc.at[idx_d[1]], sem_sc[1]).wait()
    plsc.subcore_barrier()
    pltpu.sync_copy(acc.at[pl.ds(s * _RPT, _RPT)],
                    out_hbm.at[c, pl.ds(s * _RPT, _RPT)])


def _sc2(h2aug, as2, ad2, srcp, dstp, zeros2):
    mesh = plsc.VectorSubcoreMesh(core_axis_name="c", subcore_axis_name="s")
    return pl.kernel(
        _sc2_body,
        out_type=jax.ShapeDtypeStruct((2, _NP, _ODP), _f32),
        mesh=mesh,
        compiler_params=pltpu.CompilerParams(use_tc_tiling_on_sc=False),
        scratch_types=[
            pltpu.VMEM_SHARED((_NP, _ODP), _f32),
            pltpu.VMEM((_CH,), _i32),
            pltpu.VMEM((_CH,), _i32),
            pltpu.VMEM((_CH,), _i32),
            pltpu.VMEM((_CH, _ODP), _f32),
            pltpu.VMEM((_CH, _ODP), _f32),
            pltpu.VMEM((_CH,), _f32),
            pltpu.VMEM((_CH,), _f32),
            pltpu.SemaphoreType.DMA,
            pltpu.SemaphoreType.DMA,
            pltpu.SemaphoreType.DMA,
            pltpu.SemaphoreType.DMA,
            pltpu.SemaphoreType.DMA,
            pltpu.SemaphoreType.DMA,
            pltpu.SemaphoreType.DMA,
        ],
    )(h2aug, as2, ad2, srcp, dstp, zeros2)


def _tcc_body(p_ref, bid_ref, b2_ref, hg_ref):
    i = pl.program_id(0)
    bn = p_ref.shape[1]
    p = p_ref[...]
    num = p[0, :, :_OD] + p[1, :, :_OD]
    den = p[0, :, _OD:_OD + 1] + p[1, :, _OD:_OD + 1] + 1e-16
    h2 = jnp.maximum(num / den + b2_ref[...], 0.0)
    gidx = i * bn + lax.broadcasted_iota(_i32, (bn, 1), 0)
    h2 = jnp.where(gidx < _N, h2, -jnp.inf)
    bids = bid_ref[0, 0]

    @pl.when(i == 0)
    def _():
        hg_ref[...] = jnp.full((_B, _OD), -jnp.inf, _f32)

    lo = jnp.min(bids)
    hi = jnp.max(bids)

    def body(b, _):
        m = jnp.max(jnp.where(bids[:, None] == b, h2, -jnp.inf),
                    axis=0, keepdims=True)
        hg_ref[pl.ds(b, 1), :] = jnp.maximum(hg_ref[pl.ds(b, 1), :], m)
        return _

    lax.fori_loop(lo, hi + 1, body, None)


def _tcc(parts, bids3, b2c):
    bn = 1024
    return pl.pallas_call(
        _tcc_body,
        grid=(_NP // bn,),
        in_specs=[
            pl.BlockSpec((2, bn, _ODP), lambda n: (0, n, 0)),
            pl.BlockSpec((1, 1, bn), lambda n: (n, 0, 0)),
            pl.BlockSpec((1, _OD), lambda n: (0, 0)),
        ],
        out_specs=pl.BlockSpec((_B, _OD), lambda n: (0, 0)),
        out_shape=jax.ShapeDtypeStruct((_B, _OD), _f32),
    )(parts, bids3, b2c)


def _tcd1_body(enc_ref, emb_ref, wf_ref, b_ref, y_ref, st_ref):
    b = pl.program_id(0)
    enc = enc_ref[0, 0]
    oh = (enc[:, None] == lax.broadcasted_iota(_i32, (_L, 26), 1)).astype(_f32)
    xt = jnp.dot(oh, emb_ref[...], preferred_element_type=_f32,
                 precision=_PREC_HI)
    ya = jnp.dot(xt, wf_ref[...], preferred_element_type=_f32,
                 precision=_PREC)
    t_out = _L - 15
    acc = ya[0:t_out, 0:32]
    for k in range(1, 16):
        acc = acc + ya[k:k + t_out, k * 32:(k + 1) * 32]
    y = acc + b_ref[...]
    y_ref[0] = y

    @pl.when(b == 0)
    def _():
        st_ref[...] = jnp.zeros_like(st_ref)

    st_ref[0:1, :] += jnp.sum(y, axis=0)[None, :]
    st_ref[1:2, :] += jnp.sum(y * y, axis=0)[None, :]


def _tcd1(enc3, emb, wf1, c1b):
    t_out = _L - 15
    return pl.pallas_call(
        _tcd1_body,
        grid=(_B,),
        in_specs=[
            pl.BlockSpec((1, 1, _L), lambda b: (b, 0, 0)),
            pl.BlockSpec((26, 128), lambda b: (0, 0)),
            pl.BlockSpec((128, 512), lambda b: (0, 0)),
            pl.BlockSpec((1, 32), lambda b: (0, 0)),
        ],
        out_specs=[
            pl.BlockSpec((1, t_out, 32), lambda b: (b, 0, 0)),
            pl.BlockSpec((2, 32), lambda b: (0, 0)),
        ],
        out_shape=[
            jax.ShapeDtypeStruct((_B, t_out, 32), _f32),
            jax.ShapeDtypeStruct((2, 32), _f32),
        ],
    )(enc3, emb, wf1, c1b)


def _bn_scale_shift(st_ref, g_ref, bb_ref, m_count):
    m = st_ref[0:1, :] * (1.0 / m_count)
    var = st_ref[1:2, :] * (1.0 / m_count) - m * m
    inv = lax.rsqrt(var + 1e-5)
    scale = g_ref[...].T * inv
    shift = bb_ref[...].T - m * scale
    return scale, shift


def _tcd23_body(cin, t_in, t_out, y_ref, st_ref, g_ref, bb_ref, wf_ref, b_ref,
                yo_ref, so_ref):
    b = pl.program_id(0)
    scale, shift = _bn_scale_shift(st_ref, g_ref, bb_ref, _B * t_in)
    x = jnp.maximum(y_ref[0] * scale + shift, 0.0)
    cout = b_ref.shape[1]
    ya = jnp.dot(x, wf_ref[...], preferred_element_type=_f32,
                 precision=_PREC)
    acc = ya[0:t_out, 0:cout]
    for k in range(1, 16):
        acc = acc + ya[k:k + t_out, k * cout:(k + 1) * cout]
    y = acc + b_ref[...]
    yo_ref[0] = y

    @pl.when(b == 0)
    def _():
        so_ref[...] = jnp.zeros_like(so_ref)

    so_ref[0:1, :] += jnp.sum(y, axis=0)[None, :]
    so_ref[1:2, :] += jnp.sum(y * y, axis=0)[None, :]


def _tcd23(y, st, g, bb, wf, cb, cin, cout, t_in):
    t_out = t_in - 15
    return pl.pallas_call(
        functools.partial(_tcd23_body, cin, t_in, t_out),
        grid=(_B,),
        in_specs=[
            pl.BlockSpec((1, t_in, cin), lambda b: (b, 0, 0)),
            pl.BlockSpec((2, cin), lambda b: (0, 0)),
            pl.BlockSpec((cin, 1), lambda b: (0, 0)),
            pl.BlockSpec((cin, 1), lambda b: (0, 0)),
            pl.BlockSpec((cin, 16 * cout), lambda b: (0, 0)),
            pl.BlockSpec((1, cout), lambda b: (0, 0)),
        ],
        out_specs=[
            pl.BlockSpec((1, t_out, cout), lambda b: (b, 0, 0)),
            pl.BlockSpec((2, cout), lambda b: (0, 0)),
        ],
        out_shape=[
            jax.ShapeDtypeStruct((_B, t_out, cout), _f32),
            jax.ShapeDtypeStruct((2, cout), _f32),
        ],
    )(y, st, g, bb, wf, cb)


def _tce_body(y_ref, st_ref, g_ref, bb_ref, w_ref, b_ref, xo_ref, so_ref):
    b = pl.program_id(0)
    t_in = y_ref.shape[1]
    scale, shift = _bn_scale_shift(st_ref, g_ref, bb_ref, _B * t_in)
    x = jnp.maximum(y_ref[0] * scale + shift, 0.0)
    pmax = jnp.max(x, axis=0)[None, :]
    xt = jnp.dot(pmax, w_ref[...], preferred_element_type=_f32,
                 precision=_PREC) + b_ref[...]
    xo_ref[0] = xt

    @pl.when(b == 0)
    def _():
        so_ref[...] = jnp.zeros_like(so_ref)

    so_ref[0:1, :] += xt
    so_ref[1:2, :] += xt * xt


def _tce(y3, st3, g, bb, w, fb):
    t_in = y3.shape[1]
    return pl.pallas_call(
        _tce_body,
        grid=(_B,),
        in_specs=[
            pl.BlockSpec((1, t_in, 96), lambda b: (b, 0, 0)),
            pl.BlockSpec((2, 96), lambda b: (0, 0)),
            pl.BlockSpec((96, 1), lambda b: (0, 0)),
            pl.BlockSpec((96, 1), lambda b: (0, 0)),
            pl.BlockSpec((96, _OD), lambda b: (0, 0)),
            pl.BlockSpec((1, _OD), lambda b: (0, 0)),
        ],
        out_specs=[
            pl.BlockSpec((1, 1, _OD), lambda b: (b, 0, 0)),
            pl.BlockSpec((2, _OD), lambda b: (0, 0)),
        ],
        out_shape=[
            jax.ShapeDtypeStruct((_B, 1, _OD), _f32),
            jax.ShapeDtypeStruct((2, _OD), _f32),
        ],
    )(y3, st3, g, bb, w, fb)


def _tcf_body(hg_ref, xt_ref, st_ref, g_ref, bb_ref, fgw_ref, fgb_ref,
              f1w_ref, f1b_ref, f2w_ref, f2b_ref, ow_ref, ob_ref, o_ref):
    hg = hg_ref[...]
    hg = jnp.where(hg > -1e30, hg, 0.0)
    xg = jnp.maximum(jnp.dot(hg, fgw_ref[...], preferred_element_type=_f32,
                             precision=_PREC) + fgb_ref[...], 0.0)
    xt = xt_ref[...].reshape(_B, _OD)
    m = st_ref[0:1, :] * (1.0 / _B)
    var = st_ref[1:2, :] * (1.0 / _B) - m * m
    inv = lax.rsqrt(var + 1e-5)
    xtn = jnp.maximum((xt - m) * inv * g_ref[...] + bb_ref[...], 0.0)
    xc = jnp.concatenate([xg, xtn], axis=1)
    y = jnp.maximum(jnp.dot(xc, f1w_ref[...], preferred_element_type=_f32,
                            precision=_PREC) + f1b_ref[...], 0.0)
    y = jnp.maximum(jnp.dot(y, f2w_ref[...], preferred_element_type=_f32,
                            precision=_PREC) + f2b_ref[...], 0.0)
    o_ref[...] = jnp.dot(y, ow_ref[...], preferred_element_type=_f32,
                         precision=_PREC) + ob_ref[...]


def _tcf(hg, xt3, stf, bnfg, bnfb, fgw, fgb, f1w, f1b, f2w, f2b, ow, ob):
    return pl.pallas_call(
        _tcf_body,
        out_shape=jax.ShapeDtypeStruct((_B, 1), _f32),
    )(hg, xt3, stf, bnfg, bnfb, fgw, fgb, f1w, f1b, f2w, f2b, ow, ob)


def kernel(x, edge_index, batch, target_encoding, W1, att_src1, att_dst1, b1,
           W2, att_src2, att_dst2, b2, fc_g1_w, fc_g1_b, emb_xt, conv1_w,
           conv1_b, bn1_g, bn1_b, conv2_w, conv2_b, bn2_g, bn2_b, conv3_w,
           conv3_b, bn3_g, bn3_b, fc_xt_w, fc_xt_b, bnf_g, bnf_b, fc1_w,
           fc1_b, fc2_w, fc2_b, out_w, out_b):
    loop = jnp.arange(_N, dtype=_i32)
    padv = jnp.arange(_EPAD - _ER, dtype=_i32) % _N
    srcp = jnp.concatenate([edge_index[0].astype(_i32), loop, padv])
    dstp = jnp.concatenate([edge_index[1].astype(_i32), loop, padv])

    w1h = jnp.pad(W1.reshape(_F, _H, _F).transpose(1, 0, 2),
                  ((0, 0), (0, 0), (0, _FP - _F)))
    asw = jnp.pad(att_src1, ((0, 0), (0, _FP - _F)))[:, None, :]
    adw = jnp.pad(att_dst1, ((0, 0), (0, _FP - _F)))[:, None, :]
    xp = jnp.pad(x, ((0, _NP - _N), (0, 0)))
    h1, as1, ad1 = _tca(xp, w1h, asw, adw)

    zeros1 = jnp.zeros((_RPT, _FP), _f32)
    out1 = _sc1(h1.reshape(_H * _NP, _FP), as1.reshape(-1), ad1.reshape(-1),
                srcp, dstp, zeros1)

    w2p = jnp.pad(W2.reshape(_H, _F, _OD),
                  ((0, 0), (0, _FP - _F), (0, 0))).reshape(_H * _FP, _OD)
    b1p = jnp.pad(b1.reshape(_H, _F),
                  ((0, 0), (0, _FP - _F))).reshape(1, _H * _FP)
    h2aug, as2, ad2 = _tcb(out1, w2p, b1p, att_src2, att_dst2)

    zeros2 = jnp.zeros((_RPT, _ODP), _f32)
    parts = _sc2(h2aug, as2.reshape(-1), ad2.reshape(-1), srcp, dstp, zeros2)

    batchp = jnp.pad(batch.astype(_i32), (0, _NP - _N), mode='edge')
    bids3 = batchp.reshape(_NP // 1024, 1, 1024)
    hg = _tcc(parts, bids3, b2.reshape(1, _OD))

    enc3 = target_encoding.astype(_i32).reshape(_B, 1, _L)
    wf1 = conv1_w.transpose(1, 2, 0).reshape(128, 16 * 32)
    wf2 = conv2_w.transpose(1, 2, 0).reshape(32, 16 * 64)
    wf3 = conv3_w.transpose(1, 2, 0).reshape(64, 16 * 96)
    y1, st1 = _tcd1(enc3, emb_xt, wf1, conv1_b.reshape(1, 32))
    y2, st2 = _tcd23(y1, st1, bn1_g.reshape(32, 1), bn1_b.reshape(32, 1),
                     wf2, conv2_b.reshape(1, 64), 32, 64, 985)
    y3, st3 = _tcd23(y2, st2, bn2_g.reshape(64, 1), bn2_b.reshape(64, 1),
                     wf3, conv3_b.reshape(1, 96), 64, 96, 970)
    xt3, stf = _tce(y3, st3, bn3_g.reshape(96, 1), bn3_b.reshape(96, 1),
                    fc_xt_w, fc_xt_b.reshape(1, _OD))

    return _tcf(hg, xt3, stf, bnf_g.reshape(1, _OD), bnf_b.reshape(1, _OD),
                fc_g1_w, fc_g1_b.reshape(1, _OD), fc1_w, fc1_b.reshape(1, 1024),
                fc2_w, fc2_b.reshape(1, 256), out_w, out_b.reshape(1, 1))
